# baseline (speedup 1.0000x reference)
import functools

import jax
import jax.numpy as jnp
from jax.experimental import pallas as pl
from jax.experimental.pallas import tpu as pltpu

_E = 320000
_T = 640000
_H = 128
_BE = 4000


def _swish(v):
    return v * jax.nn.sigmoid(v)


def _dot(a, b):
    return jax.lax.dot_general(a, b, (((1,), (0,)), ((), ())),
                               preferred_element_type=jnp.float32)


def _edge_body(x_ref, rbf_ref, wji_ref, bji_ref, wkj_ref, bkj_ref, wrbf_ref,
               xji_ref, xkj_ref):
    xv = x_ref[...]
    rbf_p = _dot(rbf_ref[...], wrbf_ref[...])
    xji_ref[...] = _swish(_dot(xv, wji_ref[...]) + bji_ref[...])
    xkj_ref[...] = _swish(_dot(xv, wkj_ref[...]) + bkj_ref[...]) * rbf_p


def _edge_transform(x, rbf, Wji, bji, Wkj, bkj, W_rbf):
    grid = (_E // _BE,)
    full = lambda *s: pl.BlockSpec(s, lambda i: tuple(0 for _ in s))
    return pl.pallas_call(
        _edge_body,
        grid=grid,
        in_specs=[
            pl.BlockSpec((_BE, _H), lambda i: (i, 0)),
            pl.BlockSpec((_BE, 6), lambda i: (i, 0)),
            full(_H, _H), full(1, _H), full(_H, _H), full(1, _H), full(6, _H),
        ],
        out_specs=[pl.BlockSpec((_BE, _H), lambda i: (i, 0)),
                   pl.BlockSpec((_BE, _H), lambda i: (i, 0))],
        out_shape=[jax.ShapeDtypeStruct((_E, _H), jnp.float32),
                   jax.ShapeDtypeStruct((_E, _H), jnp.float32)],
    )(x, rbf, Wji, bji.reshape(1, _H), Wkj, bkj.reshape(1, _H), W_rbf)


_BT = 4000


def _tri_body(sbf_ref, g_ref, wsbf_ref, wbt_ref, o_ref):
    sbfp = _dot(sbf_ref[...], wsbf_ref[...])
    g = g_ref[...]
    acc = sbfp[:, 0:1] * _dot(g, wbt_ref[0])
    for j in range(1, 8):
        acc += sbfp[:, j:j + 1] * _dot(g, wbt_ref[j])
    o_ref[...] = acc


def _tri_einsum(sbf, g, W_sbf, Wbt):
    grid = (_T // _BT,)
    return pl.pallas_call(
        _tri_body,
        grid=grid,
        in_specs=[
            pl.BlockSpec((_BT, 42), lambda i: (i, 0)),
            pl.BlockSpec((_BT, _H), lambda i: (i, 0)),
            pl.BlockSpec((42, 8), lambda i: (0, 0)),
            pl.BlockSpec((8, _H, _H), lambda i: (0, 0, 0)),
        ],
        out_specs=pl.BlockSpec((_BT, _H), lambda i: (i, 0)),
        out_shape=jax.ShapeDtypeStruct((_T, _H), jnp.float32),
    )(sbf, g, W_sbf, Wbt)


def _mlp_body(h_ref, x_ref, w_ref, b_ref, o_ref):
    h = h_ref[...]
    x = x_ref[...]
    W = w_ref[...]
    B = b_ref[...]

    def lin(v, i):
        return _dot(v, W[i]) + B[i][None, :]

    h = h + _swish(lin(_swish(lin(h, 0)), 1))
    h = _swish(lin(h, 2)) + x
    h = h + _swish(lin(_swish(lin(h, 3)), 4))
    h = h + _swish(lin(_swish(lin(h, 5)), 6))
    o_ref[...] = _swish(lin(h, 7))


def _mlp_stack(h, x, Wstack, Bstack):
    grid = (_E // _BE,)
    return pl.pallas_call(
        _mlp_body,
        grid=grid,
        in_specs=[
            pl.BlockSpec((_BE, _H), lambda i: (i, 0)),
            pl.BlockSpec((_BE, _H), lambda i: (i, 0)),
            pl.BlockSpec((8, _H, _H), lambda i: (0, 0, 0)),
            pl.BlockSpec((8, _H), lambda i: (0, 0)),
        ],
        out_specs=pl.BlockSpec((_BE, _H), lambda i: (i, 0)),
        out_shape=jax.ShapeDtypeStruct((_E, _H), jnp.float32),
    )(h, x, Wstack, Bstack)


def kernel(x, rbf, sbf, idx_kj, idx_ji, W_rbf, W_sbf, Wkj, bkj, Wji, bji, Wb,
           rb0_W1, rb0_b1, rb0_W2, rb0_b2, Wlin, blin,
           ra0_W1, ra0_b1, ra0_W2, ra0_b2, ra1_W1, ra1_b1, ra1_W2, ra1_b2,
           Wout, bout):
    x_ji, x_kj = _edge_transform(x, rbf, Wji, bji, Wkj, bkj, W_rbf)
    g = jnp.take(x_kj, idx_kj, axis=0)
    t = _tri_einsum(sbf, g, W_sbf, Wb.transpose(1, 2, 0))
    agg = jax.ops.segment_sum(t, idx_ji, num_segments=_E)
    h = x_ji + agg

    Wstack = jnp.stack([rb0_W1, rb0_W2, Wlin, ra0_W1, ra0_W2,
                        ra1_W1, ra1_W2, Wout])
    Bstack = jnp.stack([rb0_b1, rb0_b2, blin, ra0_b1, ra0_b2,
                        ra1_b1, ra1_b2, bout])
    return _mlp_stack(h, x, Wstack, Bstack)
